# baseline (speedup 1.0000x reference)
import functools
import jax
import jax.numpy as jnp
from jax import lax
from jax.experimental import pallas as pl
from jax.experimental.pallas import tpu as pltpu
from jax.experimental.pallas import tpu_sc as plsc

_NC = 2
_NS = 16
_NW = _NC * _NS


def _sc_aggregate(x, src, dst, zeros_hbm, *, n, d, e):
    epw = e // _NW
    chunk = 80
    kbuf = 2
    nchunk = -(-(-(-epw // chunk)) // 2) * 2
    ngather = nchunk + 2
    pad = ngather * chunk - epw
    rfull = 640
    rlast = n - 15 * rfull
    nacc = n + 16

    src2 = src.reshape(_NW, epw)
    dst2 = dst.reshape(_NW, epw)
    fill_src = jnp.zeros((_NW, pad), jnp.int32)
    fill_dst = jnp.full((_NW, nchunk * chunk - epw), n, jnp.int32)
    src3 = jnp.concatenate([src2, fill_src], axis=1)
    dst3 = jnp.concatenate([dst2, fill_dst], axis=1).reshape(
        _NW, nchunk, chunk)

    mesh = plsc.VectorSubcoreMesh(core_axis_name="c", subcore_axis_name="s")

    @functools.partial(
        pl.kernel,
        out_type=jax.ShapeDtypeStruct((_NC, n, d), jnp.float32),
        mesh=mesh,
        scratch_types=[
            pltpu.VMEM((ngather * chunk,), jnp.int32),
            pltpu.VMEM((nchunk, chunk), jnp.int32),
            [pltpu.VMEM((chunk, d), jnp.float32)] * kbuf,
            pltpu.VMEM_SHARED((nacc, d), jnp.float32),
            [pltpu.SemaphoreType.DMA] * kbuf,
            [pltpu.SemaphoreType.DMA] * kbuf,
        ],
    )
    def agg_kernel(x_hbm, src_hbm, dst_hbm, zeros_hbm_ref, out_hbm,
                   src_v, dst_v, rows_bufs, acc_sh, gsems, ssems):
        c = lax.axis_index("c")
        s = lax.axis_index("s")
        wid = s * _NC + c

        pltpu.sync_copy(src_hbm.at[wid], src_v)
        pltpu.sync_copy(dst_hbm.at[wid], dst_v)

        row0 = s * rfull

        @pl.when(jnp.logical_and(c == 0, s < 15))
        def _():
            pltpu.sync_copy(x_hbm.at[pl.ds(row0, rfull)],
                            acc_sh.at[pl.ds(row0, rfull)])

        @pl.when(jnp.logical_and(c == 0, s == 15))
        def _():
            pltpu.sync_copy(x_hbm.at[pl.ds(15 * rfull, rlast)],
                            acc_sh.at[pl.ds(15 * rfull, rlast)])

        @pl.when(jnp.logical_and(c == 1, s < 15))
        def _():
            pltpu.sync_copy(zeros_hbm_ref,
                            acc_sh.at[pl.ds(row0, rfull)])

        @pl.when(jnp.logical_and(c == 1, s == 15))
        def _():
            pltpu.sync_copy(zeros_hbm_ref.at[pl.ds(0, rlast)],
                            acc_sh.at[pl.ds(15 * rfull, rlast)])

        plsc.subcore_barrier()

        def gath(j, b):
            pltpu.async_copy(x_hbm.at[src_v.at[pl.ds(j * chunk, chunk)]],
                             rows_bufs[b], gsems[b])

        def gwait(b):
            pltpu.make_async_copy(
                x_hbm.at[src_v.at[pl.ds(0, chunk)]], rows_bufs[b],
                gsems[b]).wait()

        gath(0, 0)

        def body(i, carry):
            j = 2 * i
            gath(j + 1, 1)
            gwait(0)
            pltpu.sync_copy(rows_bufs[0], acc_sh.at[dst_v.at[j]], add=True)
            gath(j + 2, 0)
            gwait(1)
            pltpu.sync_copy(rows_bufs[1], acc_sh.at[dst_v.at[j + 1]],
                            add=True)
            return carry

        lax.fori_loop(0, nchunk // 2, body, 0)
        gwait(0)

        plsc.subcore_barrier()

        @pl.when(s < 15)
        def _():
            pltpu.sync_copy(acc_sh.at[pl.ds(row0, rfull)],
                            out_hbm.at[c, pl.ds(row0, rfull)])

        @pl.when(s == 15)
        def _():
            pltpu.sync_copy(acc_sh.at[pl.ds(15 * rfull, rlast)],
                            out_hbm.at[c, pl.ds(15 * rfull, rlast)])

    return agg_kernel(x, src3, dst3, zeros_hbm)


def _tc_fused(agg2, w1, b1, g1, be1, w2, b2, wg, bg, batch2d, g3, b3, wh, bh,
              *, n, d, h, g, cdim, rows):
    nb = n // rows

    def body(agg_ref, w1_ref, b1_ref, g1_ref, be1_ref, w2_ref, b2_ref,
             wg_ref, bg_ref, b_ref, g3_ref, b3_ref, wh_ref, bh_ref,
             out_ref, h1_s, h2_s, st_s):
        i = pl.program_id(0)

        @pl.when(i < nb)
        def _():
            a = agg_ref[0] + agg_ref[1]
            h1 = jnp.dot(a, w1_ref[...],
                         preferred_element_type=jnp.float32) + b1_ref[...]
            h1_s[pl.ds(i * rows, rows), :] = h1

            @pl.when(i == 0)
            def _():
                st_s[...] = jnp.zeros_like(st_s)

            st_s[0:1, :] += jnp.sum(h1, axis=0, keepdims=True)
            st_s[1:2, :] += jnp.sum(h1 * h1, axis=0, keepdims=True)

        @pl.when(jnp.logical_and(i >= nb, i < 2 * nb))
        def _():
            j = i - nb
            mean = st_s[0:1, :] / n
            var = st_s[1:2, :] / n - mean * mean
            scale = g1_ref[...] * lax.rsqrt(var + 1e-5)
            shift = be1_ref[...] - mean * scale
            h1 = h1_s[pl.ds(j * rows, rows), :]
            q = jnp.maximum(h1 * scale + shift, 0.0)
            h2 = jnp.dot(q, w2_ref[...], preferred_element_type=jnp.float32)
            h2_s[pl.ds(j * rows, rows), :] = jnp.maximum(h2 + b2_ref[...],
                                                         0.0)

        @pl.when(i == 2 * nb)
        def _():
            h2v = h2_s[...]
            lg = jnp.dot(h2v, wg_ref[...],
                         preferred_element_type=jnp.float32) + bg_ref[...]
            bidx = b_ref[...]
            gid = lax.broadcasted_iota(jnp.int32, (1, g), 1)
            member = bidx == gid
            onehot = member.astype(jnp.float32)

            neg = jnp.float32(-3.0e38)
            masked = jnp.where(member, lg, neg)
            segmax = jnp.max(masked, axis=0, keepdims=True)
            nodemax = jnp.max(jnp.where(member, segmax, neg),
                              axis=1, keepdims=True)
            ex = jnp.exp(lg - nodemax)
            segsum = jnp.sum(onehot * ex, axis=0, keepdims=True)
            nodesum = jnp.sum(jnp.where(member, segsum, 0.0),
                              axis=1, keepdims=True)
            attn = ex / (nodesum + 1e-16)

            weighted = attn * h2v
            pooled = lax.dot_general(
                onehot, weighted,
                dimension_numbers=(((0,), (0,)), ((), ())),
                preferred_element_type=jnp.float32)

            mu = jnp.mean(pooled, axis=0, keepdims=True)
            var = jnp.mean((pooled - mu) ** 2, axis=0, keepdims=True)
            z = (pooled - mu) / jnp.sqrt(var + 1e-5)
            z = z * g3_ref[...] + b3_ref[...]

            z = jnp.dot(z, wh_ref[...], preferred_element_type=jnp.float32)
            z = z + bh_ref[...]
            m = jnp.max(z, axis=1, keepdims=True)
            lse = m + jnp.log(jnp.sum(jnp.exp(z - m), axis=1, keepdims=True))
            out_ref[...] = z - lse

    cfix = lambda i: (0, 0)
    return pl.pallas_call(
        body,
        grid=(2 * nb + 1,),
        in_specs=[
            pl.BlockSpec((2, rows, d), lambda i: (0, jnp.minimum(i, nb - 1),
                                                  0)),
            pl.BlockSpec((d, h), cfix),
            pl.BlockSpec((1, h), cfix),
            pl.BlockSpec((1, h), cfix),
            pl.BlockSpec((1, h), cfix),
            pl.BlockSpec((h, h), cfix),
            pl.BlockSpec((1, h), cfix),
            pl.BlockSpec((h, 1), cfix),
            pl.BlockSpec((1, 1), cfix),
            pl.BlockSpec((n, 1), cfix),
            pl.BlockSpec((1, h), cfix),
            pl.BlockSpec((1, h), cfix),
            pl.BlockSpec((h, cdim), cfix),
            pl.BlockSpec((1, cdim), cfix),
        ],
        out_specs=pl.BlockSpec((g, cdim), cfix),
        out_shape=jax.ShapeDtypeStruct((g, cdim), jnp.float32),
        scratch_shapes=[
            pltpu.VMEM((n, h), jnp.float32),
            pltpu.VMEM((n, h), jnp.float32),
            pltpu.VMEM((8, h), jnp.float32),
        ],
    )(agg2, w1, b1, g1, be1, w2, b2, wg, bg, batch2d, g3, b3, wh, bh)


def kernel(x, edge_index, batch, W1, b1, g1, be1, W2, b2, Wg, bg, g3, b3,
           Wh, bh):
    n, d = x.shape
    e = edge_index.shape[1]
    h = W1.shape[1]
    g = 64
    cdim = Wh.shape[1]

    src = edge_index[0]
    dst = edge_index[1]
    zeros_hbm = jnp.zeros((640, d), dtype=jnp.float32)

    agg2 = _sc_aggregate(x, src, dst, zeros_hbm, n=n, d=d, e=e)

    out = _tc_fused(agg2, W1, b1.reshape(1, h), g1.reshape(1, h),
                    be1.reshape(1, h), W2, b2.reshape(1, h), Wg,
                    bg.reshape(1, 1), batch.reshape(n, 1).astype(jnp.int32),
                    g3.reshape(1, h), b3.reshape(1, h), Wh,
                    bh.reshape(1, cdim),
                    n=n, d=d, h=h, g=g, cdim=cdim, rows=2000)
    return out


# speedup vs baseline: 1.3890x; 1.3890x over previous
import functools
import jax
import jax.numpy as jnp
from jax import lax
from jax.experimental import pallas as pl
from jax.experimental.pallas import tpu as pltpu
from jax.experimental.pallas import tpu_sc as plsc

_NC = 2
_NS = 16
_NW = _NC * _NS


def _sc_aggregate(x, src, dst, zeros_hbm, *, n, d, e):
    epw = e // _NW
    chunk = 80
    kbuf = 1
    nchunk = -(-epw // chunk)
    ngather = nchunk
    pad = ngather * chunk - epw
    rfull = 640
    rlast = n - 15 * rfull
    nacc = n + 16

    src2 = src.reshape(_NW, epw)
    dst2 = dst.reshape(_NW, epw)
    fill_src = jnp.zeros((_NW, pad), jnp.int32)
    fill_dst = jnp.full((_NW, nchunk * chunk - epw), n, jnp.int32)
    src3 = jnp.concatenate([src2, fill_src], axis=1)
    dst3 = jnp.concatenate([dst2, fill_dst], axis=1).reshape(
        _NW, nchunk, chunk)

    mesh = plsc.VectorSubcoreMesh(core_axis_name="c", subcore_axis_name="s")

    @functools.partial(
        pl.kernel,
        out_type=jax.ShapeDtypeStruct((_NC, n, d), jnp.float32),
        mesh=mesh,
        scratch_types=[
            pltpu.VMEM((ngather * chunk,), jnp.int32),
            pltpu.VMEM((nchunk, chunk), jnp.int32),
            [pltpu.VMEM((chunk, d), jnp.float32)] * kbuf,
            pltpu.VMEM_SHARED((nacc, d), jnp.float32),
            [pltpu.SemaphoreType.DMA] * kbuf,
            [pltpu.SemaphoreType.DMA] * kbuf,
        ],
    )
    def agg_kernel(x_hbm, src_hbm, dst_hbm, zeros_hbm_ref, out_hbm,
                   src_v, dst_v, rows_bufs, acc_sh, gsems, ssems):
        c = lax.axis_index("c")
        s = lax.axis_index("s")
        wid = s * _NC + c

        pltpu.sync_copy(src_hbm.at[wid], src_v)
        pltpu.sync_copy(dst_hbm.at[wid], dst_v)

        row0 = s * rfull

        @pl.when(jnp.logical_and(c == 0, s < 15))
        def _():
            pltpu.sync_copy(x_hbm.at[pl.ds(row0, rfull)],
                            acc_sh.at[pl.ds(row0, rfull)])

        @pl.when(jnp.logical_and(c == 0, s == 15))
        def _():
            pltpu.sync_copy(x_hbm.at[pl.ds(15 * rfull, rlast)],
                            acc_sh.at[pl.ds(15 * rfull, rlast)])

        @pl.when(jnp.logical_and(c == 1, s < 15))
        def _():
            pltpu.sync_copy(zeros_hbm_ref,
                            acc_sh.at[pl.ds(row0, rfull)])

        @pl.when(jnp.logical_and(c == 1, s == 15))
        def _():
            pltpu.sync_copy(zeros_hbm_ref.at[pl.ds(0, rlast)],
                            acc_sh.at[pl.ds(15 * rfull, rlast)])

        plsc.subcore_barrier()

        def body(j, carry):
            pltpu.async_copy(x_hbm.at[src_v.at[pl.ds(j * chunk, chunk)]],
                             rows_bufs[0], gsems[0]).wait()
            pltpu.sync_copy(rows_bufs[0], acc_sh.at[dst_v.at[j]], add=True)
            return carry

        lax.fori_loop(0, nchunk, body, 0)

        plsc.subcore_barrier()

        @pl.when(s < 15)
        def _():
            pltpu.sync_copy(acc_sh.at[pl.ds(row0, rfull)],
                            out_hbm.at[c, pl.ds(row0, rfull)])

        @pl.when(s == 15)
        def _():
            pltpu.sync_copy(acc_sh.at[pl.ds(15 * rfull, rlast)],
                            out_hbm.at[c, pl.ds(15 * rfull, rlast)])

    return agg_kernel(x, src3, dst3, zeros_hbm)


def _tc_fused(agg2, w1, b1, g1, be1, w2, b2, wg, bg, batch2d, g3, b3, wh, bh,
              *, n, d, h, g, cdim, rows):
    nb = n // rows

    def body(agg_ref, w1_ref, b1_ref, g1_ref, be1_ref, w2_ref, b2_ref,
             wg_ref, bg_ref, b_ref, g3_ref, b3_ref, wh_ref, bh_ref,
             out_ref, h1_s, h2_s, st_s):
        i = pl.program_id(0)

        @pl.when(i < nb)
        def _():
            a = agg_ref[0] + agg_ref[1]
            h1 = jnp.dot(a, w1_ref[...],
                         preferred_element_type=jnp.float32) + b1_ref[...]
            h1_s[pl.ds(i * rows, rows), :] = h1

            @pl.when(i == 0)
            def _():
                st_s[...] = jnp.zeros_like(st_s)

            st_s[0:1, :] += jnp.sum(h1, axis=0, keepdims=True)
            st_s[1:2, :] += jnp.sum(h1 * h1, axis=0, keepdims=True)

        @pl.when(jnp.logical_and(i >= nb, i < 2 * nb))
        def _():
            j = i - nb
            mean = st_s[0:1, :] / n
            var = st_s[1:2, :] / n - mean * mean
            scale = g1_ref[...] * lax.rsqrt(var + 1e-5)
            shift = be1_ref[...] - mean * scale
            h1 = h1_s[pl.ds(j * rows, rows), :]
            q = jnp.maximum(h1 * scale + shift, 0.0)
            h2 = jnp.dot(q, w2_ref[...], preferred_element_type=jnp.float32)
            h2_s[pl.ds(j * rows, rows), :] = jnp.maximum(h2 + b2_ref[...],
                                                         0.0)

        @pl.when(i == 2 * nb)
        def _():
            h2v = h2_s[...]
            lg = jnp.dot(h2v, wg_ref[...],
                         preferred_element_type=jnp.float32) + bg_ref[...]
            bidx = b_ref[...]
            gid = lax.broadcasted_iota(jnp.int32, (1, g), 1)
            member = bidx == gid
            onehot = member.astype(jnp.float32)

            neg = jnp.float32(-3.0e38)
            masked = jnp.where(member, lg, neg)
            segmax = jnp.max(masked, axis=0, keepdims=True)
            nodemax = jnp.max(jnp.where(member, segmax, neg),
                              axis=1, keepdims=True)
            ex = jnp.exp(lg - nodemax)
            segsum = jnp.sum(onehot * ex, axis=0, keepdims=True)
            nodesum = jnp.sum(jnp.where(member, segsum, 0.0),
                              axis=1, keepdims=True)
            attn = ex / (nodesum + 1e-16)

            weighted = attn * h2v
            pooled = lax.dot_general(
                onehot, weighted,
                dimension_numbers=(((0,), (0,)), ((), ())),
                preferred_element_type=jnp.float32)

            mu = jnp.mean(pooled, axis=0, keepdims=True)
            var = jnp.mean((pooled - mu) ** 2, axis=0, keepdims=True)
            z = (pooled - mu) / jnp.sqrt(var + 1e-5)
            z = z * g3_ref[...] + b3_ref[...]

            z = jnp.dot(z, wh_ref[...], preferred_element_type=jnp.float32)
            z = z + bh_ref[...]
            m = jnp.max(z, axis=1, keepdims=True)
            lse = m + jnp.log(jnp.sum(jnp.exp(z - m), axis=1, keepdims=True))
            out_ref[...] = z - lse

    cfix = lambda i: (0, 0)
    return pl.pallas_call(
        body,
        grid=(2 * nb + 1,),
        in_specs=[
            pl.BlockSpec((2, rows, d), lambda i: (0, jnp.minimum(i, nb - 1),
                                                  0)),
            pl.BlockSpec((d, h), cfix),
            pl.BlockSpec((1, h), cfix),
            pl.BlockSpec((1, h), cfix),
            pl.BlockSpec((1, h), cfix),
            pl.BlockSpec((h, h), cfix),
            pl.BlockSpec((1, h), cfix),
            pl.BlockSpec((h, 1), cfix),
            pl.BlockSpec((1, 1), cfix),
            pl.BlockSpec((n, 1), cfix),
            pl.BlockSpec((1, h), cfix),
            pl.BlockSpec((1, h), cfix),
            pl.BlockSpec((h, cdim), cfix),
            pl.BlockSpec((1, cdim), cfix),
        ],
        out_specs=pl.BlockSpec((g, cdim), cfix),
        out_shape=jax.ShapeDtypeStruct((g, cdim), jnp.float32),
        scratch_shapes=[
            pltpu.VMEM((n, h), jnp.float32),
            pltpu.VMEM((n, h), jnp.float32),
            pltpu.VMEM((8, h), jnp.float32),
        ],
    )(agg2, w1, b1, g1, be1, w2, b2, wg, bg, batch2d, g3, b3, wh, bh)


def kernel(x, edge_index, batch, W1, b1, g1, be1, W2, b2, Wg, bg, g3, b3,
           Wh, bh):
    n, d = x.shape
    e = edge_index.shape[1]
    h = W1.shape[1]
    g = 64
    cdim = Wh.shape[1]

    src = edge_index[0]
    dst = edge_index[1]
    zeros_hbm = jnp.zeros((640, d), dtype=jnp.float32)

    agg2 = _sc_aggregate(x, src, dst, zeros_hbm, n=n, d=d, e=e)

    out = _tc_fused(agg2, W1, b1.reshape(1, h), g1.reshape(1, h),
                    be1.reshape(1, h), W2, b2.reshape(1, h), Wg,
                    bg.reshape(1, 1), batch.reshape(n, 1).astype(jnp.int32),
                    g3.reshape(1, h), b3.reshape(1, h), Wh,
                    bh.reshape(1, cdim),
                    n=n, d=d, h=h, g=g, cdim=cdim, rows=2000)
    return out
